# baseline (device time: 180305 ns/iter reference)
import contextlib

import jax
import jax.numpy as jnp
from jax import lax
from jax.experimental import pallas as pl
from jax.experimental.pallas import tpu as pltpu


def _scope(name):
    return jax.named_scope(name) if _PROFILE else contextlib.nullcontext()

N_DEV = 4
BN = 2048
KH = 512
SLOT_H1 = 0
SLOT_H3 = 1
SLOT_LD = 2
_COMPUTE_ONLY = False
_PROFILE = False


def kernel(x, w_mat):
    k_glob, m_per = x.shape
    _, n_glob = w_mat.shape
    assert k_glob == N_DEV * m_per and m_per == 2 * KH
    n_tiles = n_glob // BN

    def body(
        x_hbm, w_hbm, out_ref, xb, wb,
        send_sems, recv_sems, xld_sem, wld_sems, credit_sem,
    ):
        my = lax.axis_index("i")

        def nbr_rdma(h, slot, sem, start):
            dst = (my + h) % N_DEV
            src = (my - h) % N_DEV
            return pltpu.make_async_remote_copy(
                src_ref=x_hbm.at[pl.ds(dst * m_per, m_per), :],
                dst_ref=xb.at[slot],
                send_sem=send_sems.at[sem],
                recv_sem=recv_sems.at[sem],
                device_id=(dst if start else src,),
                device_id_type=pl.DeviceIdType.MESH,
            )

        def diag_rdma(c, start):
            peer = (my + 2) % N_DEV
            return pltpu.make_async_remote_copy(
                src_ref=x_hbm.at[pl.ds(peer * m_per, m_per), pl.ds(c * KH, KH)],
                dst_ref=xb.at[SLOT_LD, :, pl.ds(c * KH, KH)],
                send_sem=send_sems.at[2 + c],
                recv_sem=recv_sems.at[2 + c],
                device_id=(peer,),
                device_id_type=pl.DeviceIdType.MESH,
            )

        if not _COMPUTE_ONLY:
            barrier_sem = pltpu.get_barrier_semaphore()
            for h in range(1, N_DEV):
                pl.semaphore_signal(
                    barrier_sem,
                    inc=1,
                    device_id=((my + h) % N_DEV,),
                    device_id_type=pl.DeviceIdType.MESH,
                )
            pl.semaphore_wait(barrier_sem, N_DEV - 1)

        xload = pltpu.make_async_copy(
            x_hbm.at[pl.ds(my * m_per, m_per), :], xb.at[SLOT_LD], xld_sem
        )
        xload.start()

        phase_a = []
        if not _COMPUTE_ONLY:
            for h, slot, sem in ((1, SLOT_H1, 0), (3, SLOT_H3, 1)):
                r = nbr_rdma(h, slot, sem, start=True)
                r.start()
                phase_a.append(r)

        steps = [
            (SLOT_LD, 0, m_per, my, None),
            (SLOT_H1, 0, m_per, (my - 1) % N_DEV,
             None if _COMPUTE_ONLY else nbr_rdma(1, SLOT_H1, 0, False)),
            (SLOT_H3, 0, m_per, (my + 1) % N_DEV,
             None if _COMPUTE_ONLY else nbr_rdma(3, SLOT_H3, 1, False)),
            (SLOT_LD, 0, KH, (my + 2) % N_DEV,
             None if _COMPUTE_ONLY else diag_rdma(0, False)),
            (SLOT_LD, KH, KH, (my + 2) % N_DEV,
             None if _COMPUTE_ONLY else diag_rdma(1, False)),
        ]

        def w_ref(step_idx, nt):
            _, koff, klen, src, _ = steps[step_idx]
            return w_hbm.at[
                pl.ds(src * m_per + koff, klen), pl.ds(nt * BN, BN)
            ]

        def start_wload(idx):
            si, nt = divmod(idx, n_tiles)
            klen = steps[si][2]
            pltpu.make_async_copy(
                w_ref(si, nt), wb.at[idx % 2, pl.ds(0, klen), :],
                wld_sems.at[idx % 2],
            ).start()

        start_wload(0)
        xload.wait()

        n_steps = len(steps) * n_tiles
        diag_b = []
        for si, (slot, koff, klen, src, recv) in enumerate(steps):
            if _COMPUTE_ONLY and si >= 3:
                koff = 0
            if not _COMPUTE_ONLY and recv is not None:
                with _scope(f"waitrecv#si={si}"):
                    recv.wait_recv()
            with _scope(f"comp#si={si}"):
                for nt in range(n_tiles):
                    idx = si * n_tiles + nt
                    if idx + 1 < n_steps:
                        start_wload(idx + 1)
                    pltpu.make_async_copy(
                        w_ref(si, nt), wb.at[idx % 2, pl.ds(0, klen), :],
                        wld_sems.at[idx % 2],
                    ).wait()
                    partial = jnp.dot(
                        xb[slot, :, pl.ds(koff, klen)],
                        wb[idx % 2, pl.ds(0, klen), :],
                        preferred_element_type=jnp.float32,
                    )
                    if si == 0:
                        out_ref[:, pl.ds(nt * BN, BN)] = partial
                    else:
                        out_ref[:, pl.ds(nt * BN, BN)] += partial
            if _COMPUTE_ONLY:
                continue
            if si == 0:
                pl.semaphore_signal(
                    credit_sem,
                    inc=1,
                    device_id=((my + 2) % N_DEV,),
                    device_id_type=pl.DeviceIdType.MESH,
                )
            if si == 2:
                with _scope("phaseB_issue"):
                    for r in phase_a:
                        r.wait_send()
                    pl.semaphore_wait(credit_sem, 1)
                    for cc in range(2):
                        r = diag_rdma(cc, start=True)
                        r.start()
                        diag_b.append(r)

        with _scope("final_waitsend"):
            for r in diag_b:
                r.wait_send()

    return pl.pallas_call(
        body,
        out_shape=jax.ShapeDtypeStruct((m_per, n_glob), jnp.float32),
        in_specs=[
            pl.BlockSpec(memory_space=pl.ANY),
            pl.BlockSpec(memory_space=pl.ANY),
        ],
        out_specs=pl.BlockSpec(memory_space=pltpu.VMEM),
        scratch_shapes=[
            pltpu.VMEM((3, m_per, m_per), jnp.float32),
            pltpu.VMEM((2, m_per, BN), jnp.float32),
            pltpu.SemaphoreType.DMA((4,)),
            pltpu.SemaphoreType.DMA((4,)),
            pltpu.SemaphoreType.DMA,
            pltpu.SemaphoreType.DMA((2,)),
            pltpu.SemaphoreType.REGULAR,
        ],
        compiler_params=pltpu.CompilerParams(
            **({} if _COMPUTE_ONLY else {"collective_id": 0}),
            vmem_limit_bytes=63 * 1024 * 1024,
        ),
    )(x, w_mat)


# device time: 140009 ns/iter; 1.2878x vs baseline; 1.2878x over previous
import contextlib

import jax
import jax.numpy as jnp
from jax import lax
from jax.experimental import pallas as pl
from jax.experimental.pallas import tpu as pltpu


def _scope(name):
    return jax.named_scope(name) if _PROFILE else contextlib.nullcontext()

N_DEV = 4
BN = 2048
KH = 512
N_CHUNK = 2
W_DEPTH = 3
HOP_SLOT = {1: 0, 3: 2, 2: 1}
LOCAL_SLOT = 3
_COMPUTE_ONLY = False
_PROFILE = False


def kernel(x, w_mat):
    k_glob, m_per = x.shape
    _, n_glob = w_mat.shape
    assert k_glob == N_DEV * m_per and m_per == N_CHUNK * KH
    n_tiles = n_glob // BN

    def body(x_hbm, w_hbm, out_ref, xb, wb, send_sems, recv_sems, xld_sem, wld_sems):
        my = lax.axis_index("i")

        def rdma(h, c, start):
            dst = (my + h) % N_DEV
            src = (my - h) % N_DEV
            return pltpu.make_async_remote_copy(
                src_ref=x_hbm.at[pl.ds(dst * m_per, m_per), pl.ds(c * KH, KH)],
                dst_ref=xb.at[HOP_SLOT[h], :, pl.ds(c * KH, KH)],
                send_sem=send_sems.at[(h - 1) * N_CHUNK + c],
                recv_sem=recv_sems.at[(h - 1) * N_CHUNK + c],
                device_id=(dst if start else src,),
                device_id_type=pl.DeviceIdType.MESH,
            )

        if not _COMPUTE_ONLY:
            barrier_sem = pltpu.get_barrier_semaphore()
            for h in range(1, N_DEV):
                pl.semaphore_signal(
                    barrier_sem,
                    inc=1,
                    device_id=((my + h) % N_DEV,),
                    device_id_type=pl.DeviceIdType.MESH,
                )
            pl.semaphore_wait(barrier_sem, N_DEV - 1)

        xload = pltpu.make_async_copy(
            x_hbm.at[pl.ds(my * m_per, m_per), :], xb.at[LOCAL_SLOT], xld_sem
        )
        xload.start()

        phase_a = []
        if not _COMPUTE_ONLY:
            for h in (1, 3):
                for c in range(N_CHUNK):
                    r = rdma(h, c, start=True)
                    r.start()
                    phase_a.append(r)

        steps = [(LOCAL_SLOT, 0, None), (LOCAL_SLOT, 1, None)]
        for h, c in [(1, 0), (3, 0), (1, 1), (3, 1), (2, 0), (2, 1)]:
            steps.append((HOP_SLOT[h], c, rdma(h, c, start=False)))

        slot_src = {
            LOCAL_SLOT: my,
            HOP_SLOT[1]: (my - 1) % N_DEV,
            HOP_SLOT[3]: (my + 1) % N_DEV,
            HOP_SLOT[2]: (my + 2) % N_DEV,
        }

        def w_ref(step_idx, nt):
            slot, c, _ = steps[step_idx]
            return w_hbm.at[
                pl.ds(slot_src[slot] * m_per + c * KH, KH), pl.ds(nt * BN, BN)
            ]

        def start_wload(idx):
            si, nt = divmod(idx, n_tiles)
            pltpu.make_async_copy(
                w_ref(si, nt), wb.at[idx % W_DEPTH], wld_sems.at[idx % W_DEPTH]
            ).start()

        n_steps = len(steps) * n_tiles
        for i in range(W_DEPTH - 1):
            start_wload(i)
        xload.wait()

        diag_b = []
        for si, (slot, c, recv) in enumerate(steps):
            if _COMPUTE_ONLY:
                slot = LOCAL_SLOT
            elif recv is not None:
                with _scope(f"waitrecv#si={si}"):
                    recv.wait_recv()
            with _scope(f"comp#si={si}"):
                for nt in range(n_tiles):
                    idx = si * n_tiles + nt
                    if idx + W_DEPTH - 1 < n_steps:
                        start_wload(idx + W_DEPTH - 1)
                    pltpu.make_async_copy(
                        w_ref(si, nt), wb.at[idx % W_DEPTH], wld_sems.at[idx % W_DEPTH]
                    ).wait()
                    partial = jnp.dot(
                        xb[slot, :, pl.ds(c * KH, KH)],
                        wb[idx % W_DEPTH],
                        preferred_element_type=jnp.float32,
                    )
                    if si == 0:
                        out_ref[:, pl.ds(nt * BN, BN)] = partial
                    else:
                        out_ref[:, pl.ds(nt * BN, BN)] += partial
            if si == 3 and not _COMPUTE_ONLY:
                with _scope("phaseB_issue"):
                    for r in phase_a:
                        r.wait_send()
                    for cc in range(N_CHUNK):
                        r = rdma(2, cc, start=True)
                        r.start()
                        diag_b.append(r)

        with _scope("final_waitsend"):
            for r in diag_b:
                r.wait_send()

    return pl.pallas_call(
        body,
        out_shape=jax.ShapeDtypeStruct((m_per, n_glob), jnp.float32),
        in_specs=[
            pl.BlockSpec(memory_space=pl.ANY),
            pl.BlockSpec(memory_space=pl.ANY),
        ],
        out_specs=pl.BlockSpec(memory_space=pltpu.VMEM),
        scratch_shapes=[
            pltpu.VMEM((N_DEV, m_per, m_per), jnp.float32),
            pltpu.VMEM((W_DEPTH, KH, BN), jnp.float32),
            pltpu.SemaphoreType.DMA(((N_DEV - 1) * N_CHUNK,)),
            pltpu.SemaphoreType.DMA(((N_DEV - 1) * N_CHUNK,)),
            pltpu.SemaphoreType.DMA,
            pltpu.SemaphoreType.DMA((W_DEPTH,)),
        ],
        compiler_params=pltpu.CompilerParams(
            **({} if _COMPUTE_ONLY else {"collective_id": 0}),
            vmem_limit_bytes=63 * 1024 * 1024,
        ),
    )(x, w_mat)
